# Initial kernel scaffold; baseline (speedup 1.0000x reference)
import jax
import jax.numpy as jnp
from jax.experimental import pallas as pl


def kernel(inputs, targets):
    raise NotImplementedError("write your pallas kernel here")


# speedup vs baseline: 1.0480x; 1.0480x over previous
import jax
import jax.numpy as jnp
from jax.experimental import pallas as pl
from jax.experimental.pallas import tpu as pltpu

_NCLS = 100000
_B = 1024
_CB = 2048
_NBLK = (_NCLS + _CB - 1) // _CB
_LN2 = 0.6931471805599453


def _body(x_ref, tcol_ref, trow_ref, out_ref, m_ref, s_ref, tv_ref):
    j = pl.program_id(0)

    @pl.when(j == 0)
    def _init():
        m_ref[...] = jnp.full((_B, 1), -jnp.inf, jnp.float32)
        s_ref[...] = jnp.zeros((_B, 1), jnp.float32)
        tv_ref[...] = jnp.zeros((_B, 1), jnp.float32)

    x = x_ref[...]
    col_ids = j * _CB + jax.lax.broadcasted_iota(jnp.int32, (1, _CB), 1)
    xm = jnp.where(col_ids < _NCLS, x, -jnp.inf)
    bm = jnp.max(xm, axis=1, keepdims=True)
    m_old = m_ref[...]
    m_new = jnp.maximum(m_old, bm)
    s_ref[...] = s_ref[...] * jnp.exp(m_old - m_new) + jnp.sum(
        jnp.exp(xm - m_new), axis=1, keepdims=True
    )
    m_ref[...] = m_new

    tcol = tcol_ref[...]
    hit = col_ids == tcol
    tv_ref[...] += jnp.sum(jnp.where(hit, x, 0.0), axis=1, keepdims=True)

    @pl.when(j == _NBLK - 1)
    def _fin():
        lse = m_ref[...] + jnp.log(s_ref[...])
        nll = lse - tv_ref[...]
        trow = trow_ref[...]
        cnt = jnp.sum((tcol == trow).astype(jnp.float32), axis=1, keepdims=True)
        u = 1.0 / (1.0 - jnp.exp(cnt * (-_LN2)))
        num = jnp.sum(u * nll, axis=(0, 1), keepdims=True)
        den = jnp.sum(u, axis=(0, 1), keepdims=True)
        out_ref[...] = num / den


def kernel(inputs, targets):
    x = inputs.reshape(_B, _NCLS)
    t = targets.reshape(-1).astype(jnp.int32)
    tcol = t.reshape(_B, 1)
    trow = t.reshape(1, _B)
    out = pl.pallas_call(
        _body,
        grid=(_NBLK,),
        in_specs=[
            pl.BlockSpec((_B, _CB), lambda j: (0, j)),
            pl.BlockSpec((_B, 1), lambda j: (0, 0)),
            pl.BlockSpec((1, _B), lambda j: (0, 0)),
        ],
        out_specs=pl.BlockSpec((1, 1), lambda j: (0, 0)),
        out_shape=jax.ShapeDtypeStruct((1, 1), jnp.float32),
        scratch_shapes=[
            pltpu.VMEM((_B, 1), jnp.float32),
            pltpu.VMEM((_B, 1), jnp.float32),
            pltpu.VMEM((_B, 1), jnp.float32),
        ],
        compiler_params=pltpu.CompilerParams(
            dimension_semantics=("arbitrary",),
        ),
    )(x, tcol, trow)
    return out[0, 0]


# speedup vs baseline: 1.0567x; 1.0083x over previous
import jax
import jax.numpy as jnp
from jax.experimental import pallas as pl
from jax.experimental.pallas import tpu as pltpu

_NCLS = 100000
_B = 1024
_CB = 2048
_NBLK = (_NCLS + _CB - 1) // _CB
_LN2 = 0.6931471805599453


def _body(x_ref, tcol_ref, trow_ref, out_ref, m_ref, s_ref, tv_ref):
    j = pl.program_id(0)

    @pl.when(j == 0)
    def _init():
        m_ref[...] = jnp.full((_B, 1), -jnp.inf, jnp.float32)
        s_ref[...] = jnp.zeros((_B, 1), jnp.float32)
        tv_ref[...] = jnp.zeros((_B, 1), jnp.float32)

    tcol = tcol_ref[...]

    def _update(xm, x):
        bm = jnp.max(xm, axis=1, keepdims=True)
        m_old = m_ref[...]
        m_new = jnp.maximum(m_old, bm)
        s_ref[...] = s_ref[...] * jnp.exp(m_old - m_new) + jnp.sum(
            jnp.exp(xm - m_new), axis=1, keepdims=True
        )
        m_ref[...] = m_new
        col_ids = j * _CB + jax.lax.broadcasted_iota(jnp.int32, (1, _CB), 1)
        hit = col_ids == tcol
        tv_ref[...] += jnp.sum(jnp.where(hit, x, 0.0), axis=1, keepdims=True)

    @pl.when(j < _NBLK - 1)
    def _main():
        x = x_ref[...]
        _update(x, x)

    @pl.when(j == _NBLK - 1)
    def _tail():
        x = x_ref[...]
        col_ids = j * _CB + jax.lax.broadcasted_iota(jnp.int32, (1, _CB), 1)
        _update(jnp.where(col_ids < _NCLS, x, -jnp.inf), x)

    @pl.when(j == _NBLK - 1)
    def _fin():
        lse = m_ref[...] + jnp.log(s_ref[...])
        nll = lse - tv_ref[...]
        trow = trow_ref[...]
        cnt = jnp.sum((tcol == trow).astype(jnp.float32), axis=1, keepdims=True)
        u = 1.0 / (1.0 - jnp.exp(cnt * (-_LN2)))
        num = jnp.sum(u * nll, axis=(0, 1), keepdims=True)
        den = jnp.sum(u, axis=(0, 1), keepdims=True)
        out_ref[...] = num / den


def kernel(inputs, targets):
    x = inputs.reshape(_B, _NCLS)
    t = targets.reshape(-1).astype(jnp.int32)
    tcol = t.reshape(_B, 1)
    trow = t.reshape(1, _B)
    out = pl.pallas_call(
        _body,
        grid=(_NBLK,),
        in_specs=[
            pl.BlockSpec((_B, _CB), lambda j: (0, j)),
            pl.BlockSpec((_B, 1), lambda j: (0, 0)),
            pl.BlockSpec((1, _B), lambda j: (0, 0)),
        ],
        out_specs=pl.BlockSpec((1, 1), lambda j: (0, 0)),
        out_shape=jax.ShapeDtypeStruct((1, 1), jnp.float32),
        scratch_shapes=[
            pltpu.VMEM((_B, 1), jnp.float32),
            pltpu.VMEM((_B, 1), jnp.float32),
            pltpu.VMEM((_B, 1), jnp.float32),
        ],
        compiler_params=pltpu.CompilerParams(
            dimension_semantics=("arbitrary",),
        ),
    )(x, tcol, trow)
    return out[0, 0]


# speedup vs baseline: 1.0725x; 1.0150x over previous
import jax
import jax.numpy as jnp
from jax.experimental import pallas as pl
from jax.experimental.pallas import tpu as pltpu

_NCLS = 100000
_B = 1024
_CB = 4096
_NBLK = (_NCLS + _CB - 1) // _CB
_LN2 = 0.6931471805599453


def _body(x_ref, tcol_ref, trow_ref, out_ref, m_ref, s_ref, tv_ref):
    j = pl.program_id(0)

    @pl.when(j == 0)
    def _init():
        m_ref[...] = jnp.full((_B, 1), -jnp.inf, jnp.float32)
        s_ref[...] = jnp.zeros((_B, 1), jnp.float32)
        tv_ref[...] = jnp.zeros((_B, 1), jnp.float32)

    tcol = tcol_ref[...]

    def _update(xm, x):
        bm = jnp.max(xm, axis=1, keepdims=True)
        m_old = m_ref[...]
        m_new = jnp.maximum(m_old, bm)
        s_ref[...] = s_ref[...] * jnp.exp(m_old - m_new) + jnp.sum(
            jnp.exp(xm - m_new), axis=1, keepdims=True
        )
        m_ref[...] = m_new
        col_ids = j * _CB + jax.lax.broadcasted_iota(jnp.int32, (1, _CB), 1)
        hit = col_ids == tcol
        tv_ref[...] += jnp.sum(jnp.where(hit, x, 0.0), axis=1, keepdims=True)

    @pl.when(j < _NBLK - 1)
    def _main():
        x = x_ref[...]
        _update(x, x)

    @pl.when(j == _NBLK - 1)
    def _tail():
        x = x_ref[...]
        col_ids = j * _CB + jax.lax.broadcasted_iota(jnp.int32, (1, _CB), 1)
        _update(jnp.where(col_ids < _NCLS, x, -jnp.inf), x)

    @pl.when(j == _NBLK - 1)
    def _fin():
        lse = m_ref[...] + jnp.log(s_ref[...])
        nll = lse - tv_ref[...]
        trow = trow_ref[...]
        cnt = jnp.sum((tcol == trow).astype(jnp.float32), axis=1, keepdims=True)
        u = 1.0 / (1.0 - jnp.exp(cnt * (-_LN2)))
        num = jnp.sum(u * nll, axis=(0, 1), keepdims=True)
        den = jnp.sum(u, axis=(0, 1), keepdims=True)
        out_ref[...] = num / den


def kernel(inputs, targets):
    x = inputs.reshape(_B, _NCLS)
    t = targets.reshape(-1).astype(jnp.int32)
    tcol = t.reshape(_B, 1)
    trow = t.reshape(1, _B)
    out = pl.pallas_call(
        _body,
        grid=(_NBLK,),
        in_specs=[
            pl.BlockSpec((_B, _CB), lambda j: (0, j)),
            pl.BlockSpec((_B, 1), lambda j: (0, 0)),
            pl.BlockSpec((1, _B), lambda j: (0, 0)),
        ],
        out_specs=pl.BlockSpec((1, 1), lambda j: (0, 0)),
        out_shape=jax.ShapeDtypeStruct((1, 1), jnp.float32),
        scratch_shapes=[
            pltpu.VMEM((_B, 1), jnp.float32),
            pltpu.VMEM((_B, 1), jnp.float32),
            pltpu.VMEM((_B, 1), jnp.float32),
        ],
        compiler_params=pltpu.CompilerParams(
            dimension_semantics=("arbitrary",),
        ),
    )(x, tcol, trow)
    return out[0, 0]
